# baseline (speedup 1.0000x reference)
import jax
import jax.numpy as jnp
from jax.experimental import pallas as pl
from jax.experimental.pallas import tpu as pltpu


def kernel(x, idxs):
    B, C, T, H, W = x.shape
    BC = B * C
    HW = H * W
    xr = x.reshape(BC, T, HW)
    idxs32 = idxs.astype(jnp.int32)

    def body(idx_ref, x_hbm, o_hbm, sem):
        for t in range(T):
            pltpu.make_async_copy(
                x_hbm.at[:, pl.ds(idx_ref[t], 1), :],
                o_hbm.at[:, pl.ds(t, 1), :],
                sem,
            ).start()
        for t in range(T):
            pltpu.make_async_copy(
                x_hbm.at[:, pl.ds(0, 1), :],
                o_hbm.at[:, pl.ds(0, 1), :],
                sem,
            ).wait()

    out = pl.pallas_call(
        body,
        grid_spec=pltpu.PrefetchScalarGridSpec(
            num_scalar_prefetch=1,
            grid=(1,),
            in_specs=[pl.BlockSpec(memory_space=pl.ANY)],
            out_specs=pl.BlockSpec(memory_space=pl.ANY),
            scratch_shapes=[pltpu.SemaphoreType.DMA],
        ),
        out_shape=jax.ShapeDtypeStruct((BC, T, HW), x.dtype),
    )(idxs32, xr)
    return out.reshape(B, C, T, H, W)


# speedup vs baseline: 7.4920x; 7.4920x over previous
import jax
import jax.numpy as jnp
from jax.experimental import pallas as pl
from jax.experimental.pallas import tpu as pltpu


def kernel(x, idxs):
    B, C, T, H, W = x.shape
    BC = B * C
    xr = x.reshape(BC, T, H, W)
    idxs32 = idxs.astype(jnp.int32)

    BC_BLK = 128
    grid = (BC // BC_BLK, T)

    def body(idx_ref, x_hbm, o_ref, sem):
        i = pl.program_id(0)
        t = pl.program_id(1)
        src = idx_ref[t]
        pltpu.make_async_copy(
            x_hbm.at[pl.ds(i * BC_BLK, BC_BLK), pl.ds(src, 1)],
            o_ref,
            sem,
        ).start()
        pltpu.make_async_copy(
            x_hbm.at[pl.ds(i * BC_BLK, BC_BLK), pl.ds(src, 1)],
            o_ref,
            sem,
        ).wait()

    out = pl.pallas_call(
        body,
        grid_spec=pltpu.PrefetchScalarGridSpec(
            num_scalar_prefetch=1,
            grid=grid,
            in_specs=[pl.BlockSpec(memory_space=pl.ANY)],
            out_specs=pl.BlockSpec(
                (BC_BLK, 1, H, W),
                lambda i, t, idx_ref: (i, t, 0, 0),
            ),
            scratch_shapes=[pltpu.SemaphoreType.DMA],
        ),
        out_shape=jax.ShapeDtypeStruct((BC, T, H, W), x.dtype),
    )(idxs32, xr)
    return out.reshape(B, C, T, H, W)
